# baseline (speedup 1.0000x reference)
import jax
import jax.numpy as jnp
from jax.experimental import pallas as pl


def _mm_body(x_ref, emb_ref, out_ref):
    out_ref[...] = jnp.dot(
        x_ref[...], emb_ref[...], preferred_element_type=jnp.float32
    )


def kernel(x_seq, emb):
    B, K = x_seq.shape
    H = emb.shape[1]
    BB = 256
    return pl.pallas_call(
        _mm_body,
        grid=(B // BB,),
        in_specs=[
            pl.BlockSpec((BB, K), lambda i: (i, 0)),
            pl.BlockSpec((K, H), lambda i: (0, 0)),
        ],
        out_specs=pl.BlockSpec((BB, H), lambda i: (i, 0)),
        out_shape=jax.ShapeDtypeStruct((B, H), jnp.float32),
    )(x_seq, emb)
